# baseline (speedup 1.0000x reference)
import jax
import jax.numpy as jnp
from jax.experimental import pallas as pl
from jax.experimental.pallas import tpu as pltpu

H = W = 100
CIN, C1, C2 = 3, 10, 20
K = 5
OH1 = OW1 = H - K + 1
PH1 = PW1 = OH1 // 2
OH2 = OW2 = PH1 - K + 1
PH2 = PW2 = OH2 // 2
NP1 = 512
NP2 = 512
KK1 = CIN * K * W
KK2 = K * NP1
NFEATP = PH2 * NP2
H1, NCLS = 50, 3
B_TILE = 32
B_HEAD = 128


def _conv_kernel(x_ref, r1_ref, b1_ref, r2_ref, b2_ref,
                 a2_ref, lhs1_ref, a1_ref, lhs2_ref):
    f32 = jnp.float32
    bt = x_ref.shape[0]

    for kh in range(K):
        lhs1_ref[:, :, kh * CIN * W:(kh + 1) * CIN * W] = x_ref[:, kh:kh + OH1, :]
    lhs1 = lhs1_ref[...].reshape(bt * OH1, KK1)
    y1 = jnp.dot(lhs1, r1_ref[...], preferred_element_type=f32)
    m = (jnp.maximum(y1[:, :NP1], y1[:, NP1:]) + b1_ref[...]
         ).reshape(bt, PH1, 2, NP1)
    a1_ref[...] = jnp.maximum(
        jnp.maximum(m[:, :, 0, :], m[:, :, 1, :]), 0.0).astype(jnp.bfloat16)

    for kh in range(K):
        lhs2_ref[:, :, kh * NP1:(kh + 1) * NP1] = a1_ref[:, kh:kh + OH2, :]
    lhs2 = lhs2_ref[...].reshape(bt * OH2, KK2)
    y2 = jnp.dot(lhs2, r2_ref[...], preferred_element_type=f32)
    m2 = (jnp.maximum(y2[:, :NP2], y2[:, NP2:]) + b2_ref[...]
          ).reshape(bt, PH2, 2, NP2)
    a2_ref[...] = jnp.maximum(
        jnp.maximum(m2[:, :, 0, :], m2[:, :, 1, :]), 0.0).astype(jnp.bfloat16)


def _head_kernel(f_ref, w1_ref, b1_ref, w2_ref, b2_ref, o_ref):
    f32 = jnp.float32
    h = jnp.dot(f_ref[...], w1_ref[...], preferred_element_type=f32) + b1_ref[...]
    h = jnp.maximum(h, 0.0)
    o_ref[...] = jnp.dot(h, w2_ref[...], preferred_element_type=f32) + b2_ref[...]


def _banded(w, win, wout, npad):
    k_ = jnp.arange(K)[:, None, None]
    w_ = jnp.arange(win)[None, :, None]
    p_ = jnp.arange(wout // 2)[None, None, :]
    cout, cin = w.shape[0], w.shape[1]
    ncol = cout * (wout // 2)

    def mk(parity):
        sel = (w_ == 2 * p_ + parity + k_).astype(w.dtype)
        r = jnp.einsum('ochk,kwp->hcwop', w, sel).reshape(K * cin * win, ncol)
        return jnp.pad(r, ((0, 0), (0, npad - ncol)))

    return jnp.concatenate([mk(0), mk(1)], axis=1).astype(jnp.bfloat16)


def _pad_cols(v, ncol, npad):
    return jnp.pad(v.reshape(1, ncol), ((0, 0), (0, npad - ncol)))


def kernel(x, conv1_w, conv1_b, conv2_w, conv2_b, fc1_w, fc1_b, fc2_w, fc2_b):
    n = x.shape[0]
    bt = min(B_TILE, n)
    num_tiles = -(-n // bt)
    n_pad = num_tiles * bt

    xr = jnp.transpose(x, (0, 2, 1, 3)).reshape(n, H, CIN * W).astype(jnp.bfloat16)
    if n_pad != n:
        xr = jnp.concatenate(
            [xr, jnp.zeros((n_pad - n,) + xr.shape[1:], xr.dtype)], axis=0)

    r1 = _banded(conv1_w, W, OW1, NP1)
    b1r = _pad_cols(jnp.repeat(conv1_b, PW1), C1 * PW1, NP1)
    r2 = _banded(conv2_w, PW1, OW2, NP2)
    r2 = jnp.pad(r2.reshape(K, C1 * PW1, 2 * NP2),
                 ((0, 0), (0, NP1 - C1 * PW1), (0, 0))).reshape(KK2, 2 * NP2)
    b2r = _pad_cols(jnp.repeat(conv2_b, PW2), C2 * PW2, NP2)
    fw1 = jnp.transpose(fc1_w.reshape(H1, C2, PH2, PW2), (2, 1, 3, 0))
    fw1 = jnp.pad(fw1.reshape(PH2, C2 * PW2, H1),
                  ((0, 0), (0, NP2 - C2 * PW2), (0, 0)))
    fw1 = fw1.reshape(NFEATP, H1).astype(jnp.bfloat16)
    fb1 = fc1_b.reshape(1, H1)
    fw2 = fc2_w.T
    fb2 = fc2_b.reshape(1, NCLS)

    def full(shape):
        zeros = (0,) * len(shape)
        return pl.BlockSpec(shape, lambda g: zeros)

    a2 = pl.pallas_call(
        _conv_kernel,
        out_shape=jax.ShapeDtypeStruct((n_pad, PH2, NP2), jnp.bfloat16),
        grid=(num_tiles,),
        in_specs=[
            pl.BlockSpec((bt, H, CIN * W), lambda g: (g, 0, 0)),
            full((KK1, 2 * NP1)), full((1, NP1)),
            full((KK2, 2 * NP2)), full((1, NP2)),
        ],
        out_specs=pl.BlockSpec((bt, PH2, NP2), lambda g: (g, 0, 0)),
        scratch_shapes=[
            pltpu.VMEM((bt, OH1, KK1), jnp.bfloat16),
            pltpu.VMEM((bt, PH1, NP1), jnp.bfloat16),
            pltpu.VMEM((bt, OH2, KK2), jnp.bfloat16),
        ],
        compiler_params=pltpu.CompilerParams(
            dimension_semantics=("parallel",),
            vmem_limit_bytes=56 * 1024 * 1024),
    )(xr, r1, b1r, r2, b2r)

    feat = a2.reshape(n_pad, NFEATP)
    bh = min(B_HEAD, n_pad)
    hv_tiles = -(-n_pad // bh)
    n_pad2 = hv_tiles * bh
    if n_pad2 != n_pad:
        feat = jnp.concatenate(
            [feat, jnp.zeros((n_pad2 - n_pad, NFEATP), feat.dtype)], axis=0)

    out = pl.pallas_call(
        _head_kernel,
        out_shape=jax.ShapeDtypeStruct((n_pad2, NCLS), jnp.float32),
        grid=(hv_tiles,),
        in_specs=[
            pl.BlockSpec((bh, NFEATP), lambda g: (g, 0)),
            full((NFEATP, H1)), full((1, H1)),
            full((H1, NCLS)), full((1, NCLS)),
        ],
        out_specs=pl.BlockSpec((bh, NCLS), lambda g: (g, 0)),
        compiler_params=pltpu.CompilerParams(
            dimension_semantics=("parallel",),
            vmem_limit_bytes=56 * 1024 * 1024),
    )(feat, fw1, fb1, fw2, fb2)
    return out[:n]


# speedup vs baseline: 1.0152x; 1.0152x over previous
import jax
import jax.numpy as jnp
from jax.experimental import pallas as pl
from jax.experimental.pallas import tpu as pltpu

H = W = 100
CIN, C1, C2 = 3, 10, 20
K = 5
OH1 = OW1 = H - K + 1
PH1 = PW1 = OH1 // 2
OH2 = OW2 = PH1 - K + 1
PH2 = PW2 = OH2 // 2
PH2P = 24
NP1 = 512
NP2 = 512
KK1 = CIN * K * W
KK2 = K * NP1
H1, NCLS = 50, 3
HB = PH2 * H1
B_TILE = 32
B_HEAD = 128


def _conv_kernel(x_ref, r1_ref, b1_ref, r2_ref, b2_ref,
                 a2_ref, lhs1_ref, a1_ref, lhs2_ref):
    f32 = jnp.float32
    bt = x_ref.shape[0]

    for kh in range(K):
        lhs1_ref[:, :, kh * CIN * W:(kh + 1) * CIN * W] = x_ref[:, kh:kh + OH1, :]
    lhs1 = lhs1_ref[...].reshape(bt * OH1, KK1)
    y1 = jnp.dot(lhs1, r1_ref[...], preferred_element_type=f32)
    m = (jnp.maximum(y1[:, :NP1], y1[:, NP1:]) + b1_ref[...]
         ).reshape(bt, PH1, 2, NP1)
    a1_ref[...] = jnp.maximum(
        jnp.maximum(m[:, :, 0, :], m[:, :, 1, :]), 0.0).astype(jnp.bfloat16)

    for kh in range(K):
        lhs2_ref[:, :, kh * NP1:(kh + 1) * NP1] = a1_ref[:, kh:kh + OH2, :]
    lhs2 = lhs2_ref[...].reshape(bt * OH2, KK2)
    y2 = jnp.dot(lhs2, r2_ref[...], preferred_element_type=f32)
    m2 = (jnp.maximum(y2[:, :NP2], y2[:, NP2:]) + b2_ref[...]
          ).reshape(bt, PH2, 2, NP2)
    a2_ref[:, :PH2, :] = jnp.maximum(
        jnp.maximum(m2[:, :, 0, :], m2[:, :, 1, :]), 0.0).astype(jnp.bfloat16)
    a2_ref[:, PH2:, :] = jnp.zeros((bt, PH2P - PH2, NP2), jnp.bfloat16)


def _head_kernel(f_ref, wb_ref, mask_ref, fold_ref, b1_ref, w2_ref, b2_ref,
                 o_ref):
    f32 = jnp.float32
    bh = f_ref.shape[0]
    f2 = f_ref[...].reshape(bh * PH2P, NP2)
    g = jnp.dot(f2, wb_ref[...], preferred_element_type=f32)
    g3 = g.reshape(bh, PH2P, HB) * mask_ref[...][None]
    s = jnp.sum(g3, axis=1).astype(jnp.bfloat16)
    h = jnp.dot(s, fold_ref[...], preferred_element_type=f32) + b1_ref[...]
    h = jnp.maximum(h, 0.0)
    o_ref[...] = jnp.dot(h, w2_ref[...], preferred_element_type=f32) + b2_ref[...]


def _banded(w, win, wout, npad):
    k_ = jnp.arange(K)[:, None, None]
    w_ = jnp.arange(win)[None, :, None]
    p_ = jnp.arange(wout // 2)[None, None, :]
    cout, cin = w.shape[0], w.shape[1]
    ncol = cout * (wout // 2)

    def mk(parity):
        sel = (w_ == 2 * p_ + parity + k_).astype(w.dtype)
        r = jnp.einsum('ochk,kwp->hcwop', w, sel).reshape(K * cin * win, ncol)
        return jnp.pad(r, ((0, 0), (0, npad - ncol)))

    return jnp.concatenate([mk(0), mk(1)], axis=1).astype(jnp.bfloat16)


def _pad_cols(v, ncol, npad):
    return jnp.pad(v.reshape(1, ncol), ((0, 0), (0, npad - ncol)))


def kernel(x, conv1_w, conv1_b, conv2_w, conv2_b, fc1_w, fc1_b, fc2_w, fc2_b):
    n = x.shape[0]
    bt = min(B_TILE, n)
    num_tiles = -(-n // bt)
    n_pad = num_tiles * bt

    xr = jnp.transpose(x, (0, 2, 1, 3)).reshape(n, H, CIN * W).astype(jnp.bfloat16)
    if n_pad != n:
        xr = jnp.concatenate(
            [xr, jnp.zeros((n_pad - n,) + xr.shape[1:], xr.dtype)], axis=0)

    r1 = _banded(conv1_w, W, OW1, NP1)
    b1r = _pad_cols(jnp.repeat(conv1_b, PW1), C1 * PW1, NP1)
    r2 = _banded(conv2_w, PW1, OW2, NP2)
    r2 = jnp.pad(r2.reshape(K, C1 * PW1, 2 * NP2),
                 ((0, 0), (0, NP1 - C1 * PW1), (0, 0))).reshape(KK2, 2 * NP2)
    b2r = _pad_cols(jnp.repeat(conv2_b, PW2), C2 * PW2, NP2)
    fw1 = jnp.transpose(fc1_w.reshape(H1, C2, PH2, PW2), (2, 1, 3, 0))
    fw1 = jnp.pad(fw1.reshape(PH2, C2 * PW2, H1),
                  ((0, 0), (0, NP2 - C2 * PW2), (0, 0)))
    wb = jnp.transpose(fw1, (1, 0, 2)).reshape(NP2, HB).astype(jnp.bfloat16)
    rr = jnp.arange(PH2P)[:, None]
    cc = jnp.arange(HB)[None, :]
    mask = (cc // H1 == rr).astype(jnp.float32)
    fold = (cc.T % H1 == jnp.arange(H1)[None, :]).astype(jnp.bfloat16)
    fb1 = fc1_b.reshape(1, H1)
    fw2 = fc2_w.T
    fb2 = fc2_b.reshape(1, NCLS)

    def full(shape):
        zeros = (0,) * len(shape)
        return pl.BlockSpec(shape, lambda g: zeros)

    a2 = pl.pallas_call(
        _conv_kernel,
        out_shape=jax.ShapeDtypeStruct((n_pad, PH2P, NP2), jnp.bfloat16),
        grid=(num_tiles,),
        in_specs=[
            pl.BlockSpec((bt, H, CIN * W), lambda g: (g, 0, 0)),
            full((KK1, 2 * NP1)), full((1, NP1)),
            full((KK2, 2 * NP2)), full((1, NP2)),
        ],
        out_specs=pl.BlockSpec((bt, PH2P, NP2), lambda g: (g, 0, 0)),
        scratch_shapes=[
            pltpu.VMEM((bt, OH1, KK1), jnp.bfloat16),
            pltpu.VMEM((bt, PH1, NP1), jnp.bfloat16),
            pltpu.VMEM((bt, OH2, KK2), jnp.bfloat16),
        ],
        compiler_params=pltpu.CompilerParams(
            dimension_semantics=("parallel",),
            vmem_limit_bytes=56 * 1024 * 1024),
    )(xr, r1, b1r, r2, b2r)

    bh = min(B_HEAD, n_pad)
    hv_tiles = -(-n_pad // bh)
    n_pad2 = hv_tiles * bh
    if n_pad2 != n_pad:
        a2 = jnp.concatenate(
            [a2, jnp.zeros((n_pad2 - n_pad, PH2P, NP2), a2.dtype)], axis=0)

    out = pl.pallas_call(
        _head_kernel,
        out_shape=jax.ShapeDtypeStruct((n_pad2, NCLS), jnp.float32),
        grid=(hv_tiles,),
        in_specs=[
            pl.BlockSpec((bh, PH2P, NP2), lambda g: (g, 0, 0)),
            full((NP2, HB)), full((PH2P, HB)), full((HB, H1)),
            full((1, H1)), full((H1, NCLS)), full((1, NCLS)),
        ],
        out_specs=pl.BlockSpec((bh, NCLS), lambda g: (g, 0)),
        compiler_params=pltpu.CompilerParams(
            dimension_semantics=("parallel",),
            vmem_limit_bytes=56 * 1024 * 1024),
    )(a2, wb, mask, fold, fb1, fw2, fb2)
    return out[:n]


# speedup vs baseline: 1.0726x; 1.0566x over previous
import jax
import jax.numpy as jnp
from jax.experimental import pallas as pl
from jax.experimental.pallas import tpu as pltpu

H = W = 100
CIN, C1, C2 = 3, 10, 20
K = 5
OH1 = OW1 = H - K + 1
PH1 = PW1 = OH1 // 2
OH2 = OW2 = PH1 - K + 1
PH2 = PW2 = OH2 // 2
PH2P = 24
NP1 = 512
NP2 = 512
KK1 = CIN * K * W
KK2 = K * NP1
H1, NCLS = 50, 3
HB = PH2 * H1
B_TILE = 16
B_HEAD = 128


def _conv_kernel(x_ref, r1_ref, b1_ref, r2_ref, b2_ref,
                 a2_ref, lhs1_ref, a1_ref, lhs2_ref):
    f32 = jnp.float32
    bt = x_ref.shape[0]

    for kh in range(K):
        lhs1_ref[:, :, kh * CIN * W:(kh + 1) * CIN * W] = x_ref[:, kh:kh + OH1, :]
    lhs1 = lhs1_ref[...].reshape(bt * OH1, KK1)
    y1 = jnp.dot(lhs1, r1_ref[...], preferred_element_type=f32)
    m = (jnp.maximum(y1[:, :NP1], y1[:, NP1:]) + b1_ref[...]
         ).reshape(bt, PH1, 2, NP1)
    a1_ref[...] = jnp.maximum(
        jnp.maximum(m[:, :, 0, :], m[:, :, 1, :]), 0.0).astype(jnp.bfloat16)

    for kh in range(K):
        lhs2_ref[:, :, kh * NP1:(kh + 1) * NP1] = a1_ref[:, kh:kh + OH2, :]
    lhs2 = lhs2_ref[...].reshape(bt * OH2, KK2)
    y2 = jnp.dot(lhs2, r2_ref[...], preferred_element_type=f32)
    m2 = (jnp.maximum(y2[:, :NP2], y2[:, NP2:]) + b2_ref[...]
          ).reshape(bt, PH2, 2, NP2)
    a2_ref[:, :PH2, :] = jnp.maximum(
        jnp.maximum(m2[:, :, 0, :], m2[:, :, 1, :]), 0.0).astype(jnp.bfloat16)
    a2_ref[:, PH2:, :] = jnp.zeros((bt, PH2P - PH2, NP2), jnp.bfloat16)


def _head_kernel(f_ref, wb_ref, mask_ref, fold_ref, b1_ref, w2_ref, b2_ref,
                 o_ref):
    f32 = jnp.float32
    bh = f_ref.shape[0]
    f2 = f_ref[...].reshape(bh * PH2P, NP2)
    g = jnp.dot(f2, wb_ref[...], preferred_element_type=f32)
    g3 = g.reshape(bh, PH2P, HB) * mask_ref[...][None]
    s = jnp.sum(g3, axis=1).astype(jnp.bfloat16)
    h = jnp.dot(s, fold_ref[...], preferred_element_type=f32) + b1_ref[...]
    h = jnp.maximum(h, 0.0)
    o_ref[...] = jnp.dot(h, w2_ref[...], preferred_element_type=f32) + b2_ref[...]


def _banded(w, win, wout, npad):
    k_ = jnp.arange(K)[:, None, None]
    w_ = jnp.arange(win)[None, :, None]
    p_ = jnp.arange(wout // 2)[None, None, :]
    cout, cin = w.shape[0], w.shape[1]
    ncol = cout * (wout // 2)

    def mk(parity):
        sel = (w_ == 2 * p_ + parity + k_).astype(w.dtype)
        r = jnp.einsum('ochk,kwp->hcwop', w, sel).reshape(K * cin * win, ncol)
        return jnp.pad(r, ((0, 0), (0, npad - ncol)))

    return jnp.concatenate([mk(0), mk(1)], axis=1).astype(jnp.bfloat16)


def _pad_cols(v, ncol, npad):
    return jnp.pad(v.reshape(1, ncol), ((0, 0), (0, npad - ncol)))


def kernel(x, conv1_w, conv1_b, conv2_w, conv2_b, fc1_w, fc1_b, fc2_w, fc2_b):
    n = x.shape[0]
    bt = min(B_TILE, n)
    num_tiles = -(-n // bt)
    n_pad = num_tiles * bt

    xr = jnp.transpose(x, (0, 2, 1, 3)).reshape(n, H, CIN * W).astype(jnp.bfloat16)
    if n_pad != n:
        xr = jnp.concatenate(
            [xr, jnp.zeros((n_pad - n,) + xr.shape[1:], xr.dtype)], axis=0)

    r1 = _banded(conv1_w, W, OW1, NP1)
    b1r = _pad_cols(jnp.repeat(conv1_b, PW1), C1 * PW1, NP1)
    r2 = _banded(conv2_w, PW1, OW2, NP2)
    r2 = jnp.pad(r2.reshape(K, C1 * PW1, 2 * NP2),
                 ((0, 0), (0, NP1 - C1 * PW1), (0, 0))).reshape(KK2, 2 * NP2)
    b2r = _pad_cols(jnp.repeat(conv2_b, PW2), C2 * PW2, NP2)
    fw1 = jnp.transpose(fc1_w.reshape(H1, C2, PH2, PW2), (2, 1, 3, 0))
    fw1 = jnp.pad(fw1.reshape(PH2, C2 * PW2, H1),
                  ((0, 0), (0, NP2 - C2 * PW2), (0, 0)))
    wb = jnp.transpose(fw1, (1, 0, 2)).reshape(NP2, HB).astype(jnp.bfloat16)
    rr = jnp.arange(PH2P)[:, None]
    cc = jnp.arange(HB)[None, :]
    mask = (cc // H1 == rr).astype(jnp.float32)
    fold = (cc.T % H1 == jnp.arange(H1)[None, :]).astype(jnp.bfloat16)
    fb1 = fc1_b.reshape(1, H1)
    fw2 = fc2_w.T
    fb2 = fc2_b.reshape(1, NCLS)

    def full(shape):
        zeros = (0,) * len(shape)
        return pl.BlockSpec(shape, lambda g: zeros)

    a2 = pl.pallas_call(
        _conv_kernel,
        out_shape=jax.ShapeDtypeStruct((n_pad, PH2P, NP2), jnp.bfloat16),
        grid=(num_tiles,),
        in_specs=[
            pl.BlockSpec((bt, H, CIN * W), lambda g: (g, 0, 0)),
            full((KK1, 2 * NP1)), full((1, NP1)),
            full((KK2, 2 * NP2)), full((1, NP2)),
        ],
        out_specs=pl.BlockSpec((bt, PH2P, NP2), lambda g: (g, 0, 0)),
        scratch_shapes=[
            pltpu.VMEM((bt, OH1, KK1), jnp.bfloat16),
            pltpu.VMEM((bt, PH1, NP1), jnp.bfloat16),
            pltpu.VMEM((bt, OH2, KK2), jnp.bfloat16),
        ],
        compiler_params=pltpu.CompilerParams(
            dimension_semantics=("parallel",),
            vmem_limit_bytes=56 * 1024 * 1024),
    )(xr, r1, b1r, r2, b2r)

    bh = min(B_HEAD, n_pad)
    hv_tiles = -(-n_pad // bh)
    n_pad2 = hv_tiles * bh
    if n_pad2 != n_pad:
        a2 = jnp.concatenate(
            [a2, jnp.zeros((n_pad2 - n_pad, PH2P, NP2), a2.dtype)], axis=0)

    out = pl.pallas_call(
        _head_kernel,
        out_shape=jax.ShapeDtypeStruct((n_pad2, NCLS), jnp.float32),
        grid=(hv_tiles,),
        in_specs=[
            pl.BlockSpec((bh, PH2P, NP2), lambda g: (g, 0, 0)),
            full((NP2, HB)), full((PH2P, HB)), full((HB, H1)),
            full((1, H1)), full((H1, NCLS)), full((1, NCLS)),
        ],
        out_specs=pl.BlockSpec((bh, NCLS), lambda g: (g, 0)),
        compiler_params=pltpu.CompilerParams(
            dimension_semantics=("parallel",),
            vmem_limit_bytes=56 * 1024 * 1024),
    )(a2, wb, mask, fold, fb1, fw2, fb2)
    return out[:n]


# speedup vs baseline: 1.0752x; 1.0024x over previous
import jax
import jax.numpy as jnp
from jax.experimental import pallas as pl
from jax.experimental.pallas import tpu as pltpu

H = W = 100
CIN, C1, C2 = 3, 10, 20
K = 5
OH1 = OW1 = H - K + 1
PH1 = PW1 = OH1 // 2
OH2 = OW2 = PH1 - K + 1
PH2 = PW2 = OH2 // 2
PH2P = 24
NP1 = 512
NP2 = 512
KK1 = CIN * K * W
KK2 = K * NP1
H1, NCLS = 50, 3
HB = PH2 * H1
B_TILE = 8
B_HEAD = 128


def _conv_kernel(x_ref, r1_ref, b1_ref, r2_ref, b2_ref,
                 a2_ref, lhs1_ref, a1_ref, lhs2_ref):
    f32 = jnp.float32
    bt = x_ref.shape[0]

    for kh in range(K):
        lhs1_ref[:, :, kh * CIN * W:(kh + 1) * CIN * W] = x_ref[:, kh:kh + OH1, :]
    lhs1 = lhs1_ref[...].reshape(bt * OH1, KK1)
    y1 = jnp.dot(lhs1, r1_ref[...], preferred_element_type=f32)
    m = (jnp.maximum(y1[:, :NP1], y1[:, NP1:]) + b1_ref[...]
         ).reshape(bt, PH1, 2, NP1)
    a1_ref[...] = jnp.maximum(
        jnp.maximum(m[:, :, 0, :], m[:, :, 1, :]), 0.0).astype(jnp.bfloat16)

    for kh in range(K):
        lhs2_ref[:, :, kh * NP1:(kh + 1) * NP1] = a1_ref[:, kh:kh + OH2, :]
    lhs2 = lhs2_ref[...].reshape(bt * OH2, KK2)
    y2 = jnp.dot(lhs2, r2_ref[...], preferred_element_type=f32)
    m2 = (jnp.maximum(y2[:, :NP2], y2[:, NP2:]) + b2_ref[...]
          ).reshape(bt, PH2, 2, NP2)
    a2_ref[:, :PH2, :] = jnp.maximum(
        jnp.maximum(m2[:, :, 0, :], m2[:, :, 1, :]), 0.0).astype(jnp.bfloat16)
    a2_ref[:, PH2:, :] = jnp.zeros((bt, PH2P - PH2, NP2), jnp.bfloat16)


def _head_kernel(f_ref, wb_ref, mask_ref, fold_ref, b1_ref, w2_ref, b2_ref,
                 o_ref):
    f32 = jnp.float32
    bh = f_ref.shape[0]
    f2 = f_ref[...].reshape(bh * PH2P, NP2)
    g = jnp.dot(f2, wb_ref[...], preferred_element_type=f32)
    g3 = g.reshape(bh, PH2P, HB) * mask_ref[...][None]
    s = jnp.sum(g3, axis=1).astype(jnp.bfloat16)
    h = jnp.dot(s, fold_ref[...], preferred_element_type=f32) + b1_ref[...]
    h = jnp.maximum(h, 0.0)
    o_ref[...] = jnp.dot(h, w2_ref[...], preferred_element_type=f32) + b2_ref[...]


def _banded(w, win, wout, npad):
    k_ = jnp.arange(K)[:, None, None]
    w_ = jnp.arange(win)[None, :, None]
    p_ = jnp.arange(wout // 2)[None, None, :]
    cout, cin = w.shape[0], w.shape[1]
    ncol = cout * (wout // 2)

    def mk(parity):
        sel = (w_ == 2 * p_ + parity + k_).astype(w.dtype)
        r = jnp.einsum('ochk,kwp->hcwop', w, sel).reshape(K * cin * win, ncol)
        return jnp.pad(r, ((0, 0), (0, npad - ncol)))

    return jnp.concatenate([mk(0), mk(1)], axis=1).astype(jnp.bfloat16)


def _pad_cols(v, ncol, npad):
    return jnp.pad(v.reshape(1, ncol), ((0, 0), (0, npad - ncol)))


def kernel(x, conv1_w, conv1_b, conv2_w, conv2_b, fc1_w, fc1_b, fc2_w, fc2_b):
    n = x.shape[0]
    bt = min(B_TILE, n)
    num_tiles = -(-n // bt)
    n_pad = num_tiles * bt

    xr = jnp.transpose(x, (0, 2, 1, 3)).reshape(n, H, CIN * W).astype(jnp.bfloat16)
    if n_pad != n:
        xr = jnp.concatenate(
            [xr, jnp.zeros((n_pad - n,) + xr.shape[1:], xr.dtype)], axis=0)

    r1 = _banded(conv1_w, W, OW1, NP1)
    b1r = _pad_cols(jnp.repeat(conv1_b, PW1), C1 * PW1, NP1)
    r2 = _banded(conv2_w, PW1, OW2, NP2)
    r2 = jnp.pad(r2.reshape(K, C1 * PW1, 2 * NP2),
                 ((0, 0), (0, NP1 - C1 * PW1), (0, 0))).reshape(KK2, 2 * NP2)
    b2r = _pad_cols(jnp.repeat(conv2_b, PW2), C2 * PW2, NP2)
    fw1 = jnp.transpose(fc1_w.reshape(H1, C2, PH2, PW2), (2, 1, 3, 0))
    fw1 = jnp.pad(fw1.reshape(PH2, C2 * PW2, H1),
                  ((0, 0), (0, NP2 - C2 * PW2), (0, 0)))
    wb = jnp.transpose(fw1, (1, 0, 2)).reshape(NP2, HB).astype(jnp.bfloat16)
    rr = jnp.arange(PH2P)[:, None]
    cc = jnp.arange(HB)[None, :]
    mask = (cc // H1 == rr).astype(jnp.float32)
    fold = (cc.T % H1 == jnp.arange(H1)[None, :]).astype(jnp.bfloat16)
    fb1 = fc1_b.reshape(1, H1)
    fw2 = fc2_w.T
    fb2 = fc2_b.reshape(1, NCLS)

    def full(shape):
        zeros = (0,) * len(shape)
        return pl.BlockSpec(shape, lambda g: zeros)

    a2 = pl.pallas_call(
        _conv_kernel,
        out_shape=jax.ShapeDtypeStruct((n_pad, PH2P, NP2), jnp.bfloat16),
        grid=(num_tiles,),
        in_specs=[
            pl.BlockSpec((bt, H, CIN * W), lambda g: (g, 0, 0)),
            full((KK1, 2 * NP1)), full((1, NP1)),
            full((KK2, 2 * NP2)), full((1, NP2)),
        ],
        out_specs=pl.BlockSpec((bt, PH2P, NP2), lambda g: (g, 0, 0)),
        scratch_shapes=[
            pltpu.VMEM((bt, OH1, KK1), jnp.bfloat16),
            pltpu.VMEM((bt, PH1, NP1), jnp.bfloat16),
            pltpu.VMEM((bt, OH2, KK2), jnp.bfloat16),
        ],
        compiler_params=pltpu.CompilerParams(
            dimension_semantics=("parallel",),
            vmem_limit_bytes=56 * 1024 * 1024),
    )(xr, r1, b1r, r2, b2r)

    bh = min(B_HEAD, n_pad)
    hv_tiles = -(-n_pad // bh)
    n_pad2 = hv_tiles * bh
    if n_pad2 != n_pad:
        a2 = jnp.concatenate(
            [a2, jnp.zeros((n_pad2 - n_pad, PH2P, NP2), a2.dtype)], axis=0)

    out = pl.pallas_call(
        _head_kernel,
        out_shape=jax.ShapeDtypeStruct((n_pad2, NCLS), jnp.float32),
        grid=(hv_tiles,),
        in_specs=[
            pl.BlockSpec((bh, PH2P, NP2), lambda g: (g, 0, 0)),
            full((NP2, HB)), full((PH2P, HB)), full((HB, H1)),
            full((1, H1)), full((H1, NCLS)), full((1, NCLS)),
        ],
        out_specs=pl.BlockSpec((bh, NCLS), lambda g: (g, 0)),
        compiler_params=pltpu.CompilerParams(
            dimension_semantics=("parallel",),
            vmem_limit_bytes=56 * 1024 * 1024),
    )(a2, wb, mask, fold, fb1, fw2, fb2)
    return out[:n]


# speedup vs baseline: 1.0773x; 1.0020x over previous
import jax
import jax.numpy as jnp
from jax.experimental import pallas as pl
from jax.experimental.pallas import tpu as pltpu

H = W = 100
CIN, C1, C2 = 3, 10, 20
K = 5
OH1 = OW1 = H - K + 1
PH1 = PW1 = OH1 // 2
OH2 = OW2 = PH1 - K + 1
PH2 = PW2 = OH2 // 2
PH2P = 24
NP1 = 512
NP2 = 512
KK1 = CIN * K * W
KK2 = K * NP1
H1, NCLS = 50, 3
HB = PH2 * H1
B_TILE = 8
B_HEAD = 128


def _conv_kernel(x_ref, r1_ref, b1_ref, r2_ref, b2_ref,
                 a2_ref, lhs1_ref, a1_ref, lhs2_ref):
    f32 = jnp.float32
    bt = x_ref.shape[0]

    for kh in range(K):
        lhs1_ref[:, :, kh * CIN * W:(kh + 1) * CIN * W] = x_ref[:, kh:kh + OH1, :]
    lhs1 = lhs1_ref[...].reshape(bt * OH1, KK1)
    y1 = jnp.dot(lhs1, r1_ref[...], preferred_element_type=f32)
    m = (jnp.maximum(y1[:, :NP1], y1[:, NP1:]) + b1_ref[...]
         ).reshape(bt, PH1, 2, NP1)
    a1_ref[...] = jnp.maximum(
        jnp.maximum(m[:, :, 0, :], m[:, :, 1, :]), 0.0).astype(jnp.bfloat16)

    for kh in range(K):
        lhs2_ref[:, :, kh * NP1:(kh + 1) * NP1] = a1_ref[:, kh:kh + OH2, :]
    lhs2 = lhs2_ref[...].reshape(bt * OH2, KK2)
    y2 = jnp.dot(lhs2, r2_ref[...], preferred_element_type=f32)
    m2 = (jnp.maximum(y2[:, :NP2], y2[:, NP2:]) + b2_ref[...]
          ).reshape(bt, PH2, 2, NP2)
    a2_ref[:, :PH2, :] = jnp.maximum(
        jnp.maximum(m2[:, :, 0, :], m2[:, :, 1, :]), 0.0).astype(jnp.bfloat16)
    a2_ref[:, PH2:, :] = jnp.zeros((bt, PH2P - PH2, NP2), jnp.bfloat16)


def _head_kernel(f_ref, wb_ref, mask_ref, b1_ref, w2_ref, b2_ref, o_ref):
    f32 = jnp.float32
    bh = f_ref.shape[0]
    f2 = f_ref[...].reshape(bh * PH2P, NP2)
    g = jnp.dot(f2, wb_ref[...], preferred_element_type=f32)
    g3 = g.reshape(bh, PH2P, HB) * mask_ref[...][None]
    s = jnp.sum(g3, axis=1)
    h = b1_ref[...]
    for r in range(PH2):
        h = h + s[:, r * H1:(r + 1) * H1]
    h = jnp.maximum(h, 0.0)
    o_ref[...] = jnp.dot(h, w2_ref[...], preferred_element_type=f32) + b2_ref[...]


def _banded(w, win, wout, npad):
    k_ = jnp.arange(K)[:, None, None]
    w_ = jnp.arange(win)[None, :, None]
    p_ = jnp.arange(wout // 2)[None, None, :]
    cout, cin = w.shape[0], w.shape[1]
    ncol = cout * (wout // 2)

    def mk(parity):
        sel = (w_ == 2 * p_ + parity + k_).astype(w.dtype)
        r = jnp.einsum('ochk,kwp->hcwop', w, sel).reshape(K * cin * win, ncol)
        return jnp.pad(r, ((0, 0), (0, npad - ncol)))

    return jnp.concatenate([mk(0), mk(1)], axis=1).astype(jnp.bfloat16)


def _pad_cols(v, ncol, npad):
    return jnp.pad(v.reshape(1, ncol), ((0, 0), (0, npad - ncol)))


def kernel(x, conv1_w, conv1_b, conv2_w, conv2_b, fc1_w, fc1_b, fc2_w, fc2_b):
    n = x.shape[0]
    bt = min(B_TILE, n)
    num_tiles = -(-n // bt)
    n_pad = num_tiles * bt

    xr = jnp.transpose(x, (0, 2, 1, 3)).reshape(n, H, CIN * W).astype(jnp.bfloat16)
    if n_pad != n:
        xr = jnp.concatenate(
            [xr, jnp.zeros((n_pad - n,) + xr.shape[1:], xr.dtype)], axis=0)

    r1 = _banded(conv1_w, W, OW1, NP1)
    b1r = _pad_cols(jnp.repeat(conv1_b, PW1), C1 * PW1, NP1)
    r2 = _banded(conv2_w, PW1, OW2, NP2)
    r2 = jnp.pad(r2.reshape(K, C1 * PW1, 2 * NP2),
                 ((0, 0), (0, NP1 - C1 * PW1), (0, 0))).reshape(KK2, 2 * NP2)
    b2r = _pad_cols(jnp.repeat(conv2_b, PW2), C2 * PW2, NP2)
    fw1 = jnp.transpose(fc1_w.reshape(H1, C2, PH2, PW2), (2, 1, 3, 0))
    fw1 = jnp.pad(fw1.reshape(PH2, C2 * PW2, H1),
                  ((0, 0), (0, NP2 - C2 * PW2), (0, 0)))
    wb = jnp.transpose(fw1, (1, 0, 2)).reshape(NP2, HB).astype(jnp.bfloat16)
    rr = jnp.arange(PH2P)[:, None]
    cc = jnp.arange(HB)[None, :]
    mask = (cc // H1 == rr).astype(jnp.float32)
    fb1 = fc1_b.reshape(1, H1)
    fw2 = fc2_w.T
    fb2 = fc2_b.reshape(1, NCLS)

    def full(shape):
        zeros = (0,) * len(shape)
        return pl.BlockSpec(shape, lambda g: zeros)

    a2 = pl.pallas_call(
        _conv_kernel,
        out_shape=jax.ShapeDtypeStruct((n_pad, PH2P, NP2), jnp.bfloat16),
        grid=(num_tiles,),
        in_specs=[
            pl.BlockSpec((bt, H, CIN * W), lambda g: (g, 0, 0)),
            full((KK1, 2 * NP1)), full((1, NP1)),
            full((KK2, 2 * NP2)), full((1, NP2)),
        ],
        out_specs=pl.BlockSpec((bt, PH2P, NP2), lambda g: (g, 0, 0)),
        scratch_shapes=[
            pltpu.VMEM((bt, OH1, KK1), jnp.bfloat16),
            pltpu.VMEM((bt, PH1, NP1), jnp.bfloat16),
            pltpu.VMEM((bt, OH2, KK2), jnp.bfloat16),
        ],
        compiler_params=pltpu.CompilerParams(
            dimension_semantics=("parallel",),
            vmem_limit_bytes=56 * 1024 * 1024),
    )(xr, r1, b1r, r2, b2r)

    bh = min(B_HEAD, n_pad)
    hv_tiles = -(-n_pad // bh)
    n_pad2 = hv_tiles * bh
    if n_pad2 != n_pad:
        a2 = jnp.concatenate(
            [a2, jnp.zeros((n_pad2 - n_pad, PH2P, NP2), a2.dtype)], axis=0)

    out = pl.pallas_call(
        _head_kernel,
        out_shape=jax.ShapeDtypeStruct((n_pad2, NCLS), jnp.float32),
        grid=(hv_tiles,),
        in_specs=[
            pl.BlockSpec((bh, PH2P, NP2), lambda g: (g, 0, 0)),
            full((NP2, HB)), full((PH2P, HB)),
            full((1, H1)), full((H1, NCLS)), full((1, NCLS)),
        ],
        out_specs=pl.BlockSpec((bh, NCLS), lambda g: (g, 0)),
        compiler_params=pltpu.CompilerParams(
            dimension_semantics=("parallel",),
            vmem_limit_bytes=56 * 1024 * 1024),
    )(a2, wb, mask, fb1, fw2, fb2)
    return out[:n]
